# baseline (speedup 1.0000x reference)
import functools

import jax
import jax.numpy as jnp
from jax import lax
from jax.experimental import pallas as pl
from jax.experimental.pallas import tpu as pltpu
from jax.experimental.pallas import tpu_sc as plsc

MF_DIM = 16
MLP_DIM = 32


def _sc_gather(user_ids, item_ids, gmf_user, gmf_item, mlp_user, mlp_item):
    batch = user_ids.shape[0]
    info = plsc.get_sparse_core_info()
    nw = info.num_cores * info.num_subcores
    bpw = batch // nw
    f32 = jnp.float32

    mesh = plsc.VectorSubcoreMesh(core_axis_name="c", subcore_axis_name="s")

    @functools.partial(
        pl.kernel,
        out_type=[
            jax.ShapeDtypeStruct((batch, MF_DIM), f32),
            jax.ShapeDtypeStruct((batch, MF_DIM), f32),
            jax.ShapeDtypeStruct((batch, MLP_DIM), f32),
            jax.ShapeDtypeStruct((batch, MLP_DIM), f32),
        ],
        mesh=mesh,
        compiler_params=pltpu.CompilerParams(use_tc_tiling_on_sc=False),
        scratch_types=[
            pltpu.VMEM((bpw,), jnp.int32),
            pltpu.VMEM((bpw,), jnp.int32),
            pltpu.VMEM((bpw, MF_DIM), f32),
            pltpu.VMEM((bpw, MF_DIM), f32),
            pltpu.VMEM((bpw, MLP_DIM), f32),
            pltpu.VMEM((bpw, MLP_DIM), f32),
            pltpu.SemaphoreType.DMA,
        ],
    )
    def sc_kernel(uid_hbm, iid_hbm, gu_hbm, gi_hbm, mu_hbm, mi_hbm,
                  gu_out, gi_out, mu_out, mi_out,
                  uidx, iidx, gu_v, gi_v, mu_v, mi_v, sem):
        wid = lax.axis_index("s") * info.num_cores + lax.axis_index("c")
        base = wid * bpw
        pltpu.sync_copy(uid_hbm.at[pl.ds(base, bpw)], uidx)
        pltpu.sync_copy(iid_hbm.at[pl.ds(base, bpw)], iidx)
        c1 = pltpu.async_copy(gu_hbm.at[uidx], gu_v, sem)
        c2 = pltpu.async_copy(gi_hbm.at[iidx], gi_v, sem)
        c3 = pltpu.async_copy(mu_hbm.at[uidx], mu_v, sem)
        c4 = pltpu.async_copy(mi_hbm.at[iidx], mi_v, sem)
        c1.wait()
        c2.wait()
        c3.wait()
        c4.wait()
        pltpu.sync_copy(gu_v, gu_out.at[pl.ds(base, bpw)])
        pltpu.sync_copy(gi_v, gi_out.at[pl.ds(base, bpw)])
        pltpu.sync_copy(mu_v, mu_out.at[pl.ds(base, bpw)])
        pltpu.sync_copy(mi_v, mi_out.at[pl.ds(base, bpw)])

    return sc_kernel(user_ids, item_ids, gmf_user, gmf_item,
                     mlp_user, mlp_item)


def _tc_body(gu, gi, mu, mi, w1a, w1b, b1, w2, b2, w3, b3, wg, wm, bo, out):
    hi = jax.lax.Precision.HIGHEST
    gmf = gu[:] * gi[:]
    h = jnp.dot(mu[:], w1a[:], precision=hi) + jnp.dot(mi[:], w1b[:], precision=hi)
    h = jnp.maximum(h + b1[:], 0.0)
    h = jnp.maximum(jnp.dot(h, w2[:], precision=hi) + b2[:], 0.0)
    h = jnp.maximum(jnp.dot(h, w3[:], precision=hi) + b3[:], 0.0)
    out[:] = jnp.dot(gmf, wg[:], precision=hi) + jnp.dot(h, wm[:], precision=hi) + bo[:]


def _tc_dense(gu, gi, mu, mi, W1, b1, W2, b2, W3, b3, Wout, bout):
    batch = gu.shape[0]
    blk = 2048
    grid = (batch // blk,)
    f32 = jnp.float32

    w1a = W1[:MLP_DIM]
    w1b = W1[MLP_DIM:]
    wg = Wout[:MF_DIM]
    wm = Wout[MF_DIM:]
    b1r = b1.reshape(1, -1)
    b2r = b2.reshape(1, -1)
    b3r = b3.reshape(1, -1)
    bor = bout.reshape(1, 1)

    def row_spec(width):
        return pl.BlockSpec((blk, width), lambda i: (i, 0))

    def full_spec(a):
        return pl.BlockSpec(a.shape, lambda i: (0,) * a.ndim)

    out = pl.pallas_call(
        _tc_body,
        grid=grid,
        in_specs=[
            row_spec(MF_DIM), row_spec(MF_DIM),
            row_spec(MLP_DIM), row_spec(MLP_DIM),
            full_spec(w1a), full_spec(w1b), full_spec(b1r),
            full_spec(W2), full_spec(b2r),
            full_spec(W3), full_spec(b3r),
            full_spec(wg), full_spec(wm), full_spec(bor),
        ],
        out_specs=pl.BlockSpec((blk, 1), lambda i: (i, 0)),
        out_shape=jax.ShapeDtypeStruct((batch, 1), f32),
    )(gu, gi, mu, mi, w1a, w1b, b1r, W2, b2r, W3, b3r, wg, wm, bor)
    return jnp.squeeze(out, -1)


def kernel(user_ids, item_ids, gmf_user, gmf_item, mlp_user, mlp_item,
           W1, b1, W2, b2, W3, b3, Wout, bout):
    gu, gi, mu, mi = _sc_gather(user_ids, item_ids, gmf_user, gmf_item,
                                mlp_user, mlp_item)
    return _tc_dense(gu, gi, mu, mi, W1, b1, W2, b2, W3, b3, Wout, bout)


# speedup vs baseline: 3.1963x; 3.1963x over previous
import functools

import jax
import jax.numpy as jnp
from jax import lax
from jax.experimental import pallas as pl
from jax.experimental.pallas import tpu as pltpu
from jax.experimental.pallas import tpu_sc as plsc

MF_DIM = 16
MLP_DIM = 32
RING = 16
TAIL = 64


def _sc_gather(user_ids, item_ids, gmf_user_t, gmf_item_t, mlp_user_t,
               mlp_item_t):
    batch = user_ids.shape[0]
    vocab = gmf_user_t.shape[1]
    max_aligned = (vocab // 128) * 128
    info = plsc.get_sparse_core_info()
    nw = info.num_cores * info.num_subcores
    bpw = batch // nw
    gpw = bpw // 8
    mpw = bpw // 4
    nch = bpw // RING
    f32 = jnp.float32

    mesh = plsc.VectorSubcoreMesh(core_axis_name="c", subcore_axis_name="s")

    @functools.partial(
        pl.kernel,
        out_type=[
            jax.ShapeDtypeStruct((batch // 8, 128), f32),
            jax.ShapeDtypeStruct((batch // 8, 128), f32),
            jax.ShapeDtypeStruct((batch // 4, 128), f32),
            jax.ShapeDtypeStruct((batch // 4, 128), f32),
        ],
        mesh=mesh,
        compiler_params=pltpu.CompilerParams(needs_layout_passes=False),
        scratch_types=[
            pltpu.VMEM((bpw,), jnp.int32),
            pltpu.VMEM((bpw,), jnp.int32),
            pltpu.VMEM((RING, MLP_DIM, 128), f32),
            pltpu.VMEM((gpw, 128), f32),
            pltpu.VMEM((gpw, 128), f32),
            pltpu.VMEM((mpw, 128), f32),
            pltpu.VMEM((mpw, 128), f32),
            pltpu.SemaphoreType.DMA,
        ],
    )
    def sc_kernel(uid_hbm, iid_hbm, gu_hbm, gi_hbm, mu_hbm, mi_hbm,
                  gu_out, gi_out, mu_out, mi_out,
                  uidx, iidx, ring, ogu, ogi, omu, omi, sem):
        wid = lax.axis_index("s") * info.num_cores + lax.axis_index("c")
        base = wid * bpw
        pltpu.sync_copy(uid_hbm.at[pl.ds(base, bpw)], uidx)
        pltpu.sync_copy(iid_hbm.at[pl.ds(base, bpw)], iidx)
        iot = lax.iota(jnp.int32, 16)

        def do_table(tbl, idx_ref, opack, nsub, per_row):

            def cbody(c, _):
                iv = idx_ref[pl.ds(c * RING, RING)]
                ivc = jnp.minimum(iv, max_aligned - 1)
                tcs = (ivc // 128) * 128
                lvec = jnp.minimum(iv - tcs, 127)
                handles = []
                for i in range(RING):
                    off = pl.multiple_of(tcs[i], 128)
                    handles.append(pltpu.async_copy(
                        tbl.at[:, pl.ds(off, 128)],
                        ring.at[i, pl.ds(0, nsub), :], sem))
                for i in range(RING):
                    handles[i].wait()
                    lv = jnp.full((16,), lvec[i], jnp.int32)
                    row0 = plsc.load_gather(ring.at[i], [iot, lv])
                    if per_row == 8:
                        orow = c * 2 + i // 8
                        opack[orow, pl.ds((i % 8) * 16, 16)] = row0
                    else:
                        row1 = plsc.load_gather(ring.at[i], [iot + 16, lv])
                        orow = c * 4 + i // 4
                        opack[orow, pl.ds((i % 4) * 32, 16)] = row0
                        opack[orow, pl.ds((i % 4) * 32 + 16, 16)] = row1
                return 0
            lax.fori_loop(0, nch, cbody, 0)

        do_table(gu_hbm, uidx, ogu, MF_DIM, 8)
        do_table(gi_hbm, iidx, ogi, MF_DIM, 8)
        do_table(mu_hbm, uidx, omu, MLP_DIM, 4)
        do_table(mi_hbm, iidx, omi, MLP_DIM, 4)

        pltpu.sync_copy(ogu, gu_out.at[pl.ds(wid * gpw, gpw)])
        pltpu.sync_copy(ogi, gi_out.at[pl.ds(wid * gpw, gpw)])
        pltpu.sync_copy(omu, mu_out.at[pl.ds(wid * mpw, mpw)])
        pltpu.sync_copy(omi, mi_out.at[pl.ds(wid * mpw, mpw)])

    return sc_kernel(user_ids, item_ids, gmf_user_t, gmf_item_t,
                     mlp_user_t, mlp_item_t)


def _tc_body(gu, gi, mu, mi, uids, iids,
             tgu, tgi, tmu, tmi,
             w1a, w1b, b1, w2, b2, w3, b3, wg, wm, bo, out):
    hi = jax.lax.Precision.HIGHEST
    ucol = uids[0, 0, :][:, None]
    icol = iids[0, 0, :][:, None]
    tail_iota = lax.broadcasted_iota(jnp.int32, (1, TAIL), 1) + (1000000 - TAIL)

    def fix(rows, col, tail_ref):
        onehot = (col == tail_iota).astype(jnp.float32)
        fixed = jnp.dot(onehot, tail_ref[:], precision=hi)
        return jnp.where(col >= 1000000 - TAIL, fixed, rows)

    gu_f = fix(gu[:], ucol, tgu)
    gi_f = fix(gi[:], icol, tgi)
    mu_f = fix(mu[:], ucol, tmu)
    mi_f = fix(mi[:], icol, tmi)
    gmf = gu_f * gi_f
    h = jnp.dot(mu_f, w1a[:], precision=hi) + jnp.dot(mi_f, w1b[:], precision=hi)
    h = jnp.maximum(h + b1[:], 0.0)
    h = jnp.maximum(jnp.dot(h, w2[:], precision=hi) + b2[:], 0.0)
    h = jnp.maximum(jnp.dot(h, w3[:], precision=hi) + b3[:], 0.0)
    out[:] = jnp.dot(gmf, wg[:], precision=hi) + jnp.dot(h, wm[:], precision=hi) + bo[:]


def _tc_dense(gu, gi, mu, mi, user_ids, item_ids, tails,
              W1, b1, W2, b2, W3, b3, Wout, bout):
    batch = gu.shape[0]
    blk = 2048
    grid = (batch // blk,)
    f32 = jnp.float32

    w1a = W1[:MLP_DIM]
    w1b = W1[MLP_DIM:]
    wg = Wout[:MF_DIM]
    wm = Wout[MF_DIM:]
    b1r = b1.reshape(1, -1)
    b2r = b2.reshape(1, -1)
    b3r = b3.reshape(1, -1)
    bor = bout.reshape(1, 1)
    uids3 = user_ids.reshape(batch // blk, 1, blk)
    iids3 = item_ids.reshape(batch // blk, 1, blk)

    def row_spec(width):
        return pl.BlockSpec((blk, width), lambda i: (i, 0))

    def full_spec(a):
        return pl.BlockSpec(a.shape, lambda i: (0,) * a.ndim)

    tgu, tgi, tmu, tmi = tails
    out = pl.pallas_call(
        _tc_body,
        grid=grid,
        in_specs=[
            row_spec(MF_DIM), row_spec(MF_DIM),
            row_spec(MLP_DIM), row_spec(MLP_DIM),
            pl.BlockSpec((1, 1, blk), lambda i: (i, 0, 0)),
            pl.BlockSpec((1, 1, blk), lambda i: (i, 0, 0)),
            full_spec(tgu), full_spec(tgi), full_spec(tmu), full_spec(tmi),
            full_spec(w1a), full_spec(w1b), full_spec(b1r),
            full_spec(W2), full_spec(b2r),
            full_spec(W3), full_spec(b3r),
            full_spec(wg), full_spec(wm), full_spec(bor),
        ],
        out_specs=pl.BlockSpec((blk, 1), lambda i: (i, 0)),
        out_shape=jax.ShapeDtypeStruct((batch, 1), f32),
    )(gu, gi, mu, mi, uids3, iids3, tgu, tgi, tmu, tmi,
      w1a, w1b, b1r, W2, b2r, W3, b3r, wg, wm, bor)
    return jnp.squeeze(out, -1)


def kernel(user_ids, item_ids, gmf_user, gmf_item, mlp_user, mlp_item,
           W1, b1, W2, b2, W3, b3, Wout, bout):
    batch = user_ids.shape[0]
    gup, gip, mup, mip = _sc_gather(
        user_ids, item_ids, gmf_user.T, gmf_item.T, mlp_user.T, mlp_item.T)
    gu = gup.reshape(batch, MF_DIM)
    gi = gip.reshape(batch, MF_DIM)
    mu = mup.reshape(batch, MLP_DIM)
    mi = mip.reshape(batch, MLP_DIM)
    tails = (gmf_user[-TAIL:], gmf_item[-TAIL:],
             mlp_user[-TAIL:], mlp_item[-TAIL:])
    return _tc_dense(gu, gi, mu, mi, user_ids, item_ids, tails,
                     W1, b1, W2, b2, W3, b3, Wout, bout)


# speedup vs baseline: 4.1538x; 1.2996x over previous
import functools

import jax
import jax.numpy as jnp
from jax import lax
from jax.experimental import pallas as pl
from jax.experimental.pallas import tpu as pltpu
from jax.experimental.pallas import tpu_sc as plsc

MF_DIM = 16
MLP_DIM = 32
RING = 16
TAIL = 64


def _sc_gather(user_ids, item_ids, gmf_user_t, gmf_item_t, mlp_user_t,
               mlp_item_t, tgu, tgi, tmu, tmi):
    batch = user_ids.shape[0]
    vocab = gmf_user_t.shape[1]
    max_aligned = (vocab // 128) * 128
    info = plsc.get_sparse_core_info()
    nw = info.num_cores * info.num_subcores
    bpw = batch // nw
    gpw = bpw // 8
    mpw = bpw // 4
    nch = bpw // RING
    f32 = jnp.float32

    mesh = plsc.VectorSubcoreMesh(core_axis_name="c", subcore_axis_name="s")

    @functools.partial(
        pl.kernel,
        out_type=[
            jax.ShapeDtypeStruct((batch // 8, 128), f32),
            jax.ShapeDtypeStruct((batch // 8, 128), f32),
            jax.ShapeDtypeStruct((batch // 4, 128), f32),
            jax.ShapeDtypeStruct((batch // 4, 128), f32),
        ],
        mesh=mesh,
        compiler_params=pltpu.CompilerParams(needs_layout_passes=False),
        scratch_types=[
            pltpu.VMEM((bpw,), jnp.int32),
            pltpu.VMEM((bpw,), jnp.int32),
            pltpu.VMEM((RING, MLP_DIM, 128), f32),
            pltpu.VMEM((MLP_DIM, 128), f32),
            pltpu.VMEM((gpw, 128), f32),
            pltpu.VMEM((gpw, 128), f32),
            pltpu.VMEM((mpw, 128), f32),
            pltpu.VMEM((mpw, 128), f32),
            pltpu.SemaphoreType.DMA,
            pltpu.SemaphoreType.DMA,
        ],
    )
    def sc_kernel(uid_hbm, iid_hbm, gu_hbm, gi_hbm, mu_hbm, mi_hbm,
                  tgu_hbm, tgi_hbm, tmu_hbm, tmi_hbm,
                  gu_out, gi_out, mu_out, mi_out,
                  uidx, iidx, ring, tailv, ogu, ogi, omu, omi, sema, semb):
        wid = lax.axis_index("s") * info.num_cores + lax.axis_index("c")
        base = wid * bpw
        pltpu.sync_copy(uid_hbm.at[pl.ds(base, bpw)], uidx)
        pltpu.sync_copy(iid_hbm.at[pl.ds(base, bpw)], iidx)
        iot = lax.iota(jnp.int32, 16)

        def do_table(tbl, tail_hbm, idx_ref, opack, nsub, per_row):
            pltpu.sync_copy(tail_hbm, tailv.at[pl.ds(0, nsub), :])
            npairs = bpw // RING

            def vecs(p):
                iv = idx_ref[pl.ds(p * RING, RING)]
                ivc = jnp.minimum(iv, max_aligned - 1)
                tcs = (ivc // 128) * 128
                lvec = jnp.minimum(iv - tcs, 127)
                tvec = iv - max_aligned
                return tcs, lvec, tvec

            def fire_half(p, half, sem_x):
                tcs, _, _ = vecs(p)
                for i in range(half * 8, half * 8 + 8):
                    off = pl.multiple_of(tcs[i], 128)
                    pltpu.async_copy(tbl.at[:, pl.ds(off, 128)],
                                     ring.at[i, pl.ds(0, nsub), :], sem_x)

            def extract_half(p, half, sem_x, lvec, tvec):
                for i in range(half * 8, half * 8 + 8):
                    pltpu.make_async_copy(
                        tbl.at[:, pl.ds(0, 128)],
                        ring.at[i, pl.ds(0, nsub), :], sem_x).wait()
                    lv = jnp.full((16,), lvec[i], jnp.int32)
                    tl = jnp.full((16,), tvec[i], jnp.int32)
                    is_tail = tvec[i] >= 0
                    lv = jnp.where(is_tail, tl, lv)
                    src = lax.cond(
                        is_tail,
                        lambda: plsc.load_gather(tailv.at[pl.ds(0, nsub), :],
                                                 [iot, lv]),
                        lambda: plsc.load_gather(ring.at[i], [iot, lv]))
                    if per_row == 8:
                        orow = p * 2 + i // 8
                        opack[orow, pl.ds((i % 8) * 16, 16)] = src
                    else:
                        src1 = lax.cond(
                            is_tail,
                            lambda: plsc.load_gather(
                                tailv.at[pl.ds(0, nsub), :], [iot + 16, lv]),
                            lambda: plsc.load_gather(ring.at[i],
                                                     [iot + 16, lv]))
                        orow = p * 4 + i // 4
                        opack[orow, pl.ds((i % 4) * 32, 16)] = src
                        opack[orow, pl.ds((i % 4) * 32 + 16, 16)] = src1

            fire_half(0, 0, sema)

            def pbody(p, _):
                _, lvec, tvec = vecs(p)
                fire_half(p, 1, semb)
                extract_half(p, 0, sema, lvec, tvec)

                @pl.when(p + 1 < npairs)
                def _():
                    fire_half(p + 1, 0, sema)
                extract_half(p, 1, semb, lvec, tvec)
                return 0
            lax.fori_loop(0, npairs, pbody, 0)

        do_table(gu_hbm, tgu_hbm, uidx, ogu, MF_DIM, 8)
        do_table(gi_hbm, tgi_hbm, iidx, ogi, MF_DIM, 8)
        do_table(mu_hbm, tmu_hbm, uidx, omu, MLP_DIM, 4)
        do_table(mi_hbm, tmi_hbm, iidx, omi, MLP_DIM, 4)

        pltpu.sync_copy(ogu, gu_out.at[pl.ds(wid * gpw, gpw)])
        pltpu.sync_copy(ogi, gi_out.at[pl.ds(wid * gpw, gpw)])
        pltpu.sync_copy(omu, mu_out.at[pl.ds(wid * mpw, mpw)])
        pltpu.sync_copy(omi, mi_out.at[pl.ds(wid * mpw, mpw)])

    return sc_kernel(user_ids, item_ids, gmf_user_t, gmf_item_t,
                     mlp_user_t, mlp_item_t, tgu, tgi, tmu, tmi)


def _tc_body(gu, gi, mu, mi, w1a, w1b, b1, w2, b2, w3, b3, wg, wm,
             glog, mlog):
    hi = jax.lax.Precision.HIGHEST
    gmf = gu[:] * gi[:]
    glog[:] = jnp.dot(gmf, wg[:], precision=hi)
    h = jnp.dot(mu[:], w1a[:], precision=hi) + jnp.dot(mi[:], w1b[:], precision=hi)
    h = jnp.maximum(h + b1[:], 0.0)
    h = jnp.maximum(jnp.dot(h, w2[:], precision=hi) + b2[:], 0.0)
    h = jnp.maximum(jnp.dot(h, w3[:], precision=hi) + b3[:], 0.0)
    mlog[:] = jnp.dot(h, wm[:], precision=hi)


def _tc_dense(gup, gip, mup, mip, W1, b1, W2, b2, W3, b3, Wout):
    grows = gup.shape[0]
    mrows = mup.shape[0]
    gblk = 256
    mblk = 512
    grid = (grows // gblk,)
    f32 = jnp.float32

    e4 = jnp.eye(4, dtype=f32)
    e8 = jnp.eye(8, dtype=f32)
    w1a = jnp.kron(e4, W1[:MLP_DIM])
    w1b = jnp.kron(e4, W1[MLP_DIM:])
    w2 = jnp.kron(e4, W2)
    w3 = jnp.kron(e4, W3)
    wg = jnp.kron(e8, Wout[:MF_DIM])
    wm = jnp.kron(e4, Wout[MF_DIM:])
    b1r = jnp.tile(b1, 4).reshape(1, 128)
    b2r = jnp.tile(b2, 4).reshape(1, 64)
    b3r = jnp.tile(b3, 4).reshape(1, 32)

    def full_spec(a):
        return pl.BlockSpec(a.shape, lambda i: (0,) * a.ndim)

    glog, mlog = pl.pallas_call(
        _tc_body,
        grid=grid,
        in_specs=[
            pl.BlockSpec((gblk, 128), lambda i: (i, 0)),
            pl.BlockSpec((gblk, 128), lambda i: (i, 0)),
            pl.BlockSpec((mblk, 128), lambda i: (i, 0)),
            pl.BlockSpec((mblk, 128), lambda i: (i, 0)),
            full_spec(w1a), full_spec(w1b), full_spec(b1r),
            full_spec(w2), full_spec(b2r),
            full_spec(w3), full_spec(b3r),
            full_spec(wg), full_spec(wm),
        ],
        out_specs=[
            pl.BlockSpec((gblk, 8), lambda i: (i, 0)),
            pl.BlockSpec((mblk, 4), lambda i: (i, 0)),
        ],
        out_shape=[
            jax.ShapeDtypeStruct((grows, 8), f32),
            jax.ShapeDtypeStruct((mrows, 4), f32),
        ],
    )(gup, gip, mup, mip, w1a, w1b, b1r, w2, b2r, w3, b3r, wg, wm)
    return glog, mlog


def kernel(user_ids, item_ids, gmf_user, gmf_item, mlp_user, mlp_item,
           W1, b1, W2, b2, W3, b3, Wout, bout):
    batch = user_ids.shape[0]

    def tail_pad(t):
        return jnp.pad(t[-TAIL:], ((0, 128 - TAIL), (0, 0))).T

    gup, gip, mup, mip = _sc_gather(
        user_ids, item_ids, gmf_user.T, gmf_item.T, mlp_user.T, mlp_item.T,
        tail_pad(gmf_user), tail_pad(gmf_item),
        tail_pad(mlp_user), tail_pad(mlp_item))
    glog, mlog = _tc_dense(gup, gip, mup, mip, W1, b1, W2, b2, W3, b3, Wout)
    return glog.reshape(batch) + mlog.reshape(batch) + bout[0]


# speedup vs baseline: 4.2600x; 1.0256x over previous
import functools

import jax
import jax.numpy as jnp
from jax import lax
from jax.experimental import pallas as pl
from jax.experimental.pallas import tpu as pltpu
from jax.experimental.pallas import tpu_sc as plsc

MF_DIM = 16
MLP_DIM = 32
RING = 16
TAIL = 64


def _sc_gather(user_ids, item_ids, gmf_user_t, gmf_item_t, mlp_user_t,
               mlp_item_t, tgu, tgi, tmu, tmi):
    batch = user_ids.shape[0]
    vocab = gmf_user_t.shape[1]
    max_aligned = (vocab // 128) * 128
    info = plsc.get_sparse_core_info()
    nw = info.num_cores * info.num_subcores
    bpw = batch // nw
    gpw = bpw // 8
    mpw = bpw // 4
    nch = bpw // RING
    f32 = jnp.float32

    mesh = plsc.VectorSubcoreMesh(core_axis_name="c", subcore_axis_name="s")

    @functools.partial(
        pl.kernel,
        out_type=[
            jax.ShapeDtypeStruct((batch // 8, 128), f32),
            jax.ShapeDtypeStruct((batch // 8, 128), f32),
            jax.ShapeDtypeStruct((batch // 4, 128), f32),
            jax.ShapeDtypeStruct((batch // 4, 128), f32),
        ],
        mesh=mesh,
        compiler_params=pltpu.CompilerParams(needs_layout_passes=False),
        scratch_types=[
            pltpu.VMEM((bpw,), jnp.int32),
            pltpu.VMEM((bpw,), jnp.int32),
            pltpu.VMEM((RING, MLP_DIM, 128), f32),
            pltpu.VMEM((MLP_DIM, 128), f32),
            pltpu.VMEM((gpw, 128), f32),
            pltpu.VMEM((gpw, 128), f32),
            pltpu.VMEM((mpw, 128), f32),
            pltpu.VMEM((mpw, 128), f32),
            pltpu.SemaphoreType.DMA,
            pltpu.SemaphoreType.DMA,
        ],
    )
    def sc_kernel(uid_hbm, iid_hbm, gu_hbm, gi_hbm, mu_hbm, mi_hbm,
                  tgu_hbm, tgi_hbm, tmu_hbm, tmi_hbm,
                  gu_out, gi_out, mu_out, mi_out,
                  uidx, iidx, ring, tailv, ogu, ogi, omu, omi, sema, semb):
        wid = lax.axis_index("s") * info.num_cores + lax.axis_index("c")
        base = wid * bpw
        pltpu.sync_copy(uid_hbm.at[pl.ds(base, bpw)], uidx)
        pltpu.sync_copy(iid_hbm.at[pl.ds(base, bpw)], iidx)
        iot = lax.iota(jnp.int32, 16)

        def do_table(tbl, tail_hbm, idx_ref, opack, nsub, per_row):
            pltpu.sync_copy(tail_hbm, tailv.at[pl.ds(0, nsub), :])
            npairs = bpw // RING

            def vecs(p):
                iv = idx_ref[pl.ds(p * RING, RING)]
                ivc = jnp.minimum(iv, max_aligned - 1)
                tcs = (ivc // 128) * 128
                lvec = jnp.minimum(iv - tcs, 127)
                tvec = iv - max_aligned
                return tcs, lvec, tvec

            def fire_half(p, half, sem_x):
                tcs, _, _ = vecs(p)
                for i in range(half * 8, half * 8 + 8):
                    off = pl.multiple_of(tcs[i], 128)
                    pltpu.async_copy(tbl.at[:, pl.ds(off, 128)],
                                     ring.at[i, pl.ds(0, nsub), :], sem_x)

            def extract_half(p, half, sem_x, lvec, tvec):
                for i in range(half * 8, half * 8 + 8):
                    pltpu.make_async_copy(
                        tbl.at[:, pl.ds(0, 128)],
                        ring.at[i, pl.ds(0, nsub), :], sem_x).wait()
                    lv = jnp.full((16,), lvec[i], jnp.int32)
                    tl = jnp.full((16,), tvec[i], jnp.int32)
                    is_tail = tvec[i] >= 0
                    lv = jnp.where(is_tail, tl, lv)
                    src = lax.cond(
                        is_tail,
                        lambda: plsc.load_gather(tailv.at[pl.ds(0, nsub), :],
                                                 [iot, lv]),
                        lambda: plsc.load_gather(ring.at[i], [iot, lv]))
                    if per_row == 8:
                        orow = p * 2 + i // 8
                        opack[orow, pl.ds((i % 8) * 16, 16)] = src
                    else:
                        src1 = lax.cond(
                            is_tail,
                            lambda: plsc.load_gather(
                                tailv.at[pl.ds(0, nsub), :], [iot + 16, lv]),
                            lambda: plsc.load_gather(ring.at[i],
                                                     [iot + 16, lv]))
                        orow = p * 4 + i // 4
                        opack[orow, pl.ds((i % 4) * 32, 16)] = src
                        opack[orow, pl.ds((i % 4) * 32 + 16, 16)] = src1

            fire_half(0, 0, sema)

            def pbody(p, _):
                _, lvec, tvec = vecs(p)
                fire_half(p, 1, semb)
                extract_half(p, 0, sema, lvec, tvec)

                @pl.when(p + 1 < npairs)
                def _():
                    fire_half(p + 1, 0, sema)
                extract_half(p, 1, semb, lvec, tvec)
                return 0
            lax.fori_loop(0, npairs, pbody, 0)

        do_table(gu_hbm, tgu_hbm, uidx, ogu, MF_DIM, 8)
        do_table(gi_hbm, tgi_hbm, iidx, ogi, MF_DIM, 8)
        do_table(mu_hbm, tmu_hbm, uidx, omu, MLP_DIM, 4)
        do_table(mi_hbm, tmi_hbm, iidx, omi, MLP_DIM, 4)

        pltpu.sync_copy(ogu, gu_out.at[pl.ds(wid * gpw, gpw)])
        pltpu.sync_copy(ogi, gi_out.at[pl.ds(wid * gpw, gpw)])
        pltpu.sync_copy(omu, mu_out.at[pl.ds(wid * mpw, mpw)])
        pltpu.sync_copy(omi, mi_out.at[pl.ds(wid * mpw, mpw)])

    return sc_kernel(user_ids, item_ids, gmf_user_t, gmf_item_t,
                     mlp_user_t, mlp_item_t, tgu, tgi, tmu, tmi)


def _tc_body(gu, gi, mu, mi, w1a, w1b, b1, w2, b2, w3, b3, wg, wm,
             glog, mlog):
    hi = jax.lax.Precision.DEFAULT
    gmf = gu[:] * gi[:]
    glog[:] = jnp.dot(gmf, wg[:], precision=hi)
    h = jnp.dot(mu[:], w1a[:], precision=hi) + jnp.dot(mi[:], w1b[:], precision=hi)
    h = jnp.maximum(h + b1[:], 0.0)
    h = jnp.maximum(jnp.dot(h, w2[:], precision=hi) + b2[:], 0.0)
    h = jnp.maximum(jnp.dot(h, w3[:], precision=hi) + b3[:], 0.0)
    mlog[:] = jnp.dot(h, wm[:], precision=hi)


def _tc_dense(gup, gip, mup, mip, W1, b1, W2, b2, W3, b3, Wout):
    grows = gup.shape[0]
    mrows = mup.shape[0]
    gblk = 256
    mblk = 512
    grid = (grows // gblk,)
    f32 = jnp.float32

    e4 = jnp.eye(4, dtype=f32)
    e8 = jnp.eye(8, dtype=f32)
    w1a = jnp.kron(e4, W1[:MLP_DIM])
    w1b = jnp.kron(e4, W1[MLP_DIM:])
    w2 = jnp.kron(e4, W2)
    w3 = jnp.kron(e4, W3)
    wg = jnp.kron(e8, Wout[:MF_DIM])
    wm = jnp.kron(e4, Wout[MF_DIM:])
    b1r = jnp.tile(b1, 4).reshape(1, 128)
    b2r = jnp.tile(b2, 4).reshape(1, 64)
    b3r = jnp.tile(b3, 4).reshape(1, 32)

    def full_spec(a):
        return pl.BlockSpec(a.shape, lambda i: (0,) * a.ndim)

    glog, mlog = pl.pallas_call(
        _tc_body,
        grid=grid,
        in_specs=[
            pl.BlockSpec((gblk, 128), lambda i: (i, 0)),
            pl.BlockSpec((gblk, 128), lambda i: (i, 0)),
            pl.BlockSpec((mblk, 128), lambda i: (i, 0)),
            pl.BlockSpec((mblk, 128), lambda i: (i, 0)),
            full_spec(w1a), full_spec(w1b), full_spec(b1r),
            full_spec(w2), full_spec(b2r),
            full_spec(w3), full_spec(b3r),
            full_spec(wg), full_spec(wm),
        ],
        out_specs=[
            pl.BlockSpec((gblk, 8), lambda i: (i, 0)),
            pl.BlockSpec((mblk, 4), lambda i: (i, 0)),
        ],
        out_shape=[
            jax.ShapeDtypeStruct((grows, 8), f32),
            jax.ShapeDtypeStruct((mrows, 4), f32),
        ],
    )(gup, gip, mup, mip, w1a, w1b, b1r, w2, b2r, w3, b3r, wg, wm)
    return glog, mlog


def kernel(user_ids, item_ids, gmf_user, gmf_item, mlp_user, mlp_item,
           W1, b1, W2, b2, W3, b3, Wout, bout):
    batch = user_ids.shape[0]

    def tail_pad(t):
        return jnp.pad(t[-TAIL:], ((0, 128 - TAIL), (0, 0))).T

    gup, gip, mup, mip = _sc_gather(
        user_ids, item_ids, gmf_user.T, gmf_item.T, mlp_user.T, mlp_item.T,
        tail_pad(gmf_user), tail_pad(gmf_item),
        tail_pad(mlp_user), tail_pad(mlp_item))
    glog, mlog = _tc_dense(gup, gip, mup, mip, W1, b1, W2, b2, W3, b3, Wout)
    return glog.reshape(batch) + mlog.reshape(batch) + bout[0]
